# baseline (speedup 1.0000x reference)
import functools

import jax
import jax.numpy as jnp
from jax import lax
from jax.experimental import pallas as pl
from jax.experimental.pallas import tpu as pltpu
from jax.experimental.pallas import tpu_sc as plsc

N = 10000
E = 320000
D = 128
G = 64
ATOM_VOCAB = 119
BOND_VOCAB = 5

NC = 2
NS = 16
NW = NC * NS
EPW = E // NW
C = 128
EPW_P = 10240
NCHUNK = EPW_P // C
SBATCH = 16
N_PAD = 10240
RPT = N_PAD // NS
D16 = D // 16


def _sc_aggr_body(m_hbm, idx_hbm, dst_hbm, out_hbm,
                  aggr_sh, idx_b, dst_b, rows0, rows1,
                  gsem0, gsem1, ssem0, ssem1):
    c = lax.axis_index("c")
    s = lax.axis_index("s")
    wid = c * NS + s

    def _zb(i, carry):
        for j in range(D16):
            rows0[i, pl.ds(j * 16, 16)] = jnp.zeros((16,), jnp.float32)
        return carry
    lax.fori_loop(0, C, _zb, 0)
    for k in range(RPT // C):
        pltpu.sync_copy(rows0, aggr_sh.at[pl.ds(s * RPT + k * C, C)])
    plsc.subcore_barrier()

    def _super(sb, carry):
        pltpu.sync_copy(idx_hbm.at[wid, pl.ds(sb * SBATCH, SBATCH)], idx_b)
        pltpu.sync_copy(dst_hbm.at[wid, pl.ds(sb * SBATCH, SBATCH)], dst_b)
        pltpu.async_copy(m_hbm.at[idx_b.at[0]], rows0, gsem0)

        def _pair(t, cc):
            k0 = 2 * t
            k1 = 2 * t + 1
            pltpu.make_async_copy(m_hbm.at[idx_b.at[k0]], rows0, gsem0).wait()

            @pl.when(t > 0)
            def _w1():
                pltpu.make_async_copy(
                    rows1, aggr_sh.at[dst_b.at[k1 - 2]], ssem1).wait()
            pltpu.async_copy(m_hbm.at[idx_b.at[k1]], rows1, gsem1)
            pltpu.async_copy(rows0, aggr_sh.at[dst_b.at[k0]], ssem0, add=True)
            pltpu.make_async_copy(m_hbm.at[idx_b.at[k1]], rows1, gsem1).wait()
            pltpu.make_async_copy(rows0, aggr_sh.at[dst_b.at[k0]], ssem0).wait()

            @pl.when(t < SBATCH // 2 - 1)
            def _g2():
                pltpu.async_copy(m_hbm.at[idx_b.at[k0 + 2]], rows0, gsem0)
            pltpu.async_copy(rows1, aggr_sh.at[dst_b.at[k1]], ssem1, add=True)
            return cc
        lax.fori_loop(0, SBATCH // 2, _pair, 0)
        pltpu.make_async_copy(
            rows1, aggr_sh.at[dst_b.at[SBATCH - 1]], ssem1).wait()
        return carry
    lax.fori_loop(0, NCHUNK // SBATCH, _super, 0)

    plsc.subcore_barrier()
    pltpu.sync_copy(aggr_sh.at[pl.ds(s * RPT, RPT)],
                    out_hbm.at[c, pl.ds(s * RPT, RPT)])


@functools.cache
def _make_sc_aggr():
    return pl.kernel(
        _sc_aggr_body,
        out_type=jax.ShapeDtypeStruct((NC, N_PAD, D), jnp.float32),
        mesh=plsc.VectorSubcoreMesh(core_axis_name="c", subcore_axis_name="s",
                                    num_cores=NC, num_subcores=NS),
        scratch_types=[
            pltpu.VMEM_SHARED((N_PAD, D), jnp.float32),
            pltpu.VMEM((SBATCH, C), jnp.int32),
            pltpu.VMEM((SBATCH, C), jnp.int32),
            pltpu.VMEM((C, D), jnp.float32),
            pltpu.VMEM((C, D), jnp.float32),
            pltpu.SemaphoreType.DMA,
            pltpu.SemaphoreType.DMA,
            pltpu.SemaphoreType.DMA,
            pltpu.SemaphoreType.DMA,
        ],
    )


def _msg_body(h_ref, bond_ref, out_ref):
    h = h_ref[...]
    for v in range(BOND_VOCAB):
        out_ref[:, v * D:(v + 1) * D] = jnp.maximum(h + bond_ref[v:v + 1, :],
                                                    0.0)


_msg = pl.pallas_call(
    _msg_body,
    out_shape=jax.ShapeDtypeStruct((N, BOND_VOCAB * D), jnp.float32),
)


def _embed_body(x_ref, tab_ref, out_ref):
    xv = x_ref[...]
    ids = lax.broadcasted_iota(jnp.int32, (1, ATOM_VOCAB), 1)
    oh = (xv == ids).astype(jnp.float32)
    out_ref[...] = jnp.dot(oh, tab_ref[...], preferred_element_type=jnp.float32,
                           precision=lax.Precision.HIGHEST)


_embed = pl.pallas_call(
    _embed_body,
    out_shape=jax.ShapeDtypeStruct((N, D), jnp.float32),
)


def _bn(z, g, b):
    mean = jnp.mean(z, axis=0, keepdims=True)
    var = jnp.mean((z - mean) ** 2, axis=0, keepdims=True)
    return g * (z - mean) / jnp.sqrt(var + 1e-5) + b


def _dense_core(h_ref, a_ref, w1_ref, b1_ref, g1_ref, be1_ref,
                w2_ref, b2_ref, g2_ref, be2_ref):
    z = h_ref[...] + a_ref[0, :N] + a_ref[1, :N]
    z1 = jnp.dot(z.astype(jnp.bfloat16), w1_ref[...].astype(jnp.bfloat16),
                 preferred_element_type=jnp.float32) + b1_ref[...]
    z1 = jnp.maximum(_bn(z1, g1_ref[...], be1_ref[...]), 0.0)
    z2 = jnp.dot(z1.astype(jnp.bfloat16), w2_ref[...].astype(jnp.bfloat16),
                 preferred_element_type=jnp.float32) + b2_ref[...]
    return _bn(z2, g2_ref[...], be2_ref[...])


def _dense_mid_body(h_ref, a_ref, w1_ref, b1_ref, g1_ref, be1_ref,
                    w2_ref, b2_ref, g2_ref, be2_ref, out_ref):
    out_ref[...] = jnp.maximum(
        _dense_core(h_ref, a_ref, w1_ref, b1_ref, g1_ref, be1_ref,
                    w2_ref, b2_ref, g2_ref, be2_ref), 0.0)


def _dense_last_body(h_ref, a_ref, w1_ref, b1_ref, g1_ref, be1_ref,
                     w2_ref, b2_ref, g2_ref, be2_ref, batch_ref,
                     outh_ref, outp_ref):
    hn = _dense_core(h_ref, a_ref, w1_ref, b1_ref, g1_ref, be1_ref,
                     w2_ref, b2_ref, g2_ref, be2_ref)
    outh_ref[...] = hn
    bv = batch_ref[...]
    gi = lax.broadcasted_iota(jnp.int32, (1, G), 1)
    oh = (bv == gi).astype(jnp.float32)
    xp = lax.dot_general(oh, hn, (((0,), (0,)), ((), ())),
                         preferred_element_type=jnp.float32,
                         precision=lax.Precision.HIGHEST)
    nrm = jnp.sqrt(jnp.sum(xp * xp, axis=1, keepdims=True))
    outp_ref[...] = xp / jnp.maximum(nrm, 1e-12)


_dense_mid = pl.pallas_call(
    _dense_mid_body,
    out_shape=jax.ShapeDtypeStruct((N, D), jnp.float32),
)

_dense_last = pl.pallas_call(
    _dense_last_body,
    out_shape=(jax.ShapeDtypeStruct((N, D), jnp.float32),
               jax.ShapeDtypeStruct((G, D), jnp.float32)),
)


def kernel(params, batch, x, edge_index, edge_attr):
    atom = params['atom_table']
    bond = params['bond_table']
    layers = params['layers']
    src = edge_index[0].astype(jnp.int32)
    dst = edge_index[1].astype(jnp.int32)
    ea = edge_attr[:, 0].astype(jnp.int32)
    pad = ((0, 0), (0, EPW_P - EPW))
    idx = jnp.pad((src * BOND_VOCAB + ea).reshape(NW, EPW),
                  pad).reshape(NW, NCHUNK, C)
    dstp = jnp.pad(dst.reshape(NW, EPW), pad,
                   constant_values=N_PAD - 1).reshape(NW, NCHUNK, C)
    xi = x.astype(jnp.int32)
    batchf = batch.astype(jnp.int32).reshape(N, 1)

    h = _embed(xi, atom)
    n_layers = len(layers)
    xpool = None
    for i, p in enumerate(layers):
        msg = _msg(h, bond).reshape(N * BOND_VOCAB, D)
        aggr = _make_sc_aggr()(msg, idx, dstp)
        args = (h, aggr, p['W1'], p['b1'].reshape(1, -1),
                p['g_mlp'].reshape(1, -1), p['be_mlp'].reshape(1, -1),
                p['W2'], p['b2'].reshape(1, -1),
                p['g_bn'].reshape(1, -1), p['be_bn'].reshape(1, -1))
        if i < n_layers - 1:
            h = _dense_mid(*args)
        else:
            h, xpool = _dense_last(*args, batchf)
    return (xpool, h)


# speedup vs baseline: 2.6633x; 2.6633x over previous
import functools

import jax
import jax.numpy as jnp
from jax import lax
from jax.experimental import pallas as pl
from jax.experimental.pallas import tpu as pltpu
from jax.experimental.pallas import tpu_sc as plsc

N = 10000
E = 320000
D = 128
G = 64
ATOM_VOCAB = 119
BOND_VOCAB = 5

NC = 2
NS = 16
NW = NC * NS
EPW = E // NW
C = 128
EPW_P = 10240
NCHUNK = EPW_P // C
SBATCH = 32
N_PAD = 10240
RPT = N_PAD // NS
D16 = D // 16


def _sc_aggr_body(m_hbm, idx_hbm, dst_hbm, out_hbm,
                  aggr_sh, idx_b, dst_b, rows0, rows1,
                  gsem0, gsem1, ssem0, ssem1):
    c = lax.axis_index("c")
    s = lax.axis_index("s")
    wid = c * NS + s

    def _zb(i, carry):
        for j in range(D16):
            rows0[i, pl.ds(j * 16, 16)] = jnp.zeros((16,), jnp.float32)
        return carry
    lax.fori_loop(0, C, _zb, 0)
    for k in range(RPT // C):
        pltpu.sync_copy(rows0, aggr_sh.at[pl.ds(s * RPT + k * C, C)])
    plsc.subcore_barrier()

    def _super(sb, carry):
        pltpu.sync_copy(idx_hbm.at[wid, pl.ds(sb * SBATCH, SBATCH)], idx_b)
        pltpu.sync_copy(dst_hbm.at[wid, pl.ds(sb * SBATCH, SBATCH)], dst_b)
        pltpu.async_copy(m_hbm.at[idx_b.at[0]], rows0, gsem0)

        def _pair(t, cc):
            k0 = 2 * t
            k1 = 2 * t + 1
            pltpu.make_async_copy(m_hbm.at[idx_b.at[k0]], rows0, gsem0).wait()

            @pl.when(t > 0)
            def _w1():
                pltpu.make_async_copy(
                    rows1, aggr_sh.at[dst_b.at[k1 - 2]], ssem1).wait()
            pltpu.async_copy(m_hbm.at[idx_b.at[k1]], rows1, gsem1)
            pltpu.async_copy(rows0, aggr_sh.at[dst_b.at[k0]], ssem0, add=True)
            pltpu.make_async_copy(m_hbm.at[idx_b.at[k1]], rows1, gsem1).wait()
            pltpu.make_async_copy(rows0, aggr_sh.at[dst_b.at[k0]], ssem0).wait()

            @pl.when(t < SBATCH // 2 - 1)
            def _g2():
                pltpu.async_copy(m_hbm.at[idx_b.at[k0 + 2]], rows0, gsem0)
            pltpu.async_copy(rows1, aggr_sh.at[dst_b.at[k1]], ssem1, add=True)
            return cc
        lax.fori_loop(0, SBATCH // 2, _pair, 0)
        pltpu.make_async_copy(
            rows1, aggr_sh.at[dst_b.at[SBATCH - 1]], ssem1).wait()
        return carry
    lax.fori_loop(0, NCHUNK // SBATCH, _super, 0)

    plsc.subcore_barrier()
    pltpu.sync_copy(aggr_sh.at[pl.ds(s * RPT, RPT)],
                    out_hbm.at[c, pl.ds(s * RPT, RPT)])


@functools.cache
def _make_sc_aggr():
    return pl.kernel(
        _sc_aggr_body,
        out_type=jax.ShapeDtypeStruct((NC, N_PAD, D), jnp.float32),
        mesh=plsc.VectorSubcoreMesh(core_axis_name="c", subcore_axis_name="s",
                                    num_cores=NC, num_subcores=NS),
        scratch_types=[
            pltpu.VMEM_SHARED((N_PAD, D), jnp.float32),
            pltpu.VMEM((SBATCH, C), jnp.int32),
            pltpu.VMEM((SBATCH, C), jnp.int32),
            pltpu.VMEM((C, D), jnp.float32),
            pltpu.VMEM((C, D), jnp.float32),
            pltpu.SemaphoreType.DMA,
            pltpu.SemaphoreType.DMA,
            pltpu.SemaphoreType.DMA,
            pltpu.SemaphoreType.DMA,
        ],
    )


def _msg_body(h_ref, bond_ref, out_ref):
    h = h_ref[...]
    for v in range(BOND_VOCAB):
        out_ref[:, v * D:(v + 1) * D] = jnp.maximum(h + bond_ref[v:v + 1, :],
                                                    0.0)


_msg = pl.pallas_call(
    _msg_body,
    out_shape=jax.ShapeDtypeStruct((N, BOND_VOCAB * D), jnp.float32),
)


def _embed_body(x_ref, tab_ref, out_ref):
    xv = x_ref[...]
    ids = lax.broadcasted_iota(jnp.int32, (1, ATOM_VOCAB), 1)
    oh = (xv == ids).astype(jnp.float32)
    out_ref[...] = jnp.dot(oh, tab_ref[...], preferred_element_type=jnp.float32,
                           precision=lax.Precision.HIGHEST)


_embed = pl.pallas_call(
    _embed_body,
    out_shape=jax.ShapeDtypeStruct((N, D), jnp.float32),
)


def _bn(z, g, b):
    mean = jnp.mean(z, axis=0, keepdims=True)
    var = jnp.mean((z - mean) ** 2, axis=0, keepdims=True)
    return g * (z - mean) / jnp.sqrt(var + 1e-5) + b


def _dense_core(h_ref, a_ref, w1_ref, b1_ref, g1_ref, be1_ref,
                w2_ref, b2_ref, g2_ref, be2_ref):
    z = h_ref[...] + a_ref[0, :N] + a_ref[1, :N]
    z1 = jnp.dot(z.astype(jnp.bfloat16), w1_ref[...].astype(jnp.bfloat16),
                 preferred_element_type=jnp.float32) + b1_ref[...]
    z1 = jnp.maximum(_bn(z1, g1_ref[...], be1_ref[...]), 0.0)
    z2 = jnp.dot(z1.astype(jnp.bfloat16), w2_ref[...].astype(jnp.bfloat16),
                 preferred_element_type=jnp.float32) + b2_ref[...]
    return _bn(z2, g2_ref[...], be2_ref[...])


def _dense_mid_body(h_ref, a_ref, w1_ref, b1_ref, g1_ref, be1_ref,
                    w2_ref, b2_ref, g2_ref, be2_ref, out_ref):
    out_ref[...] = jnp.maximum(
        _dense_core(h_ref, a_ref, w1_ref, b1_ref, g1_ref, be1_ref,
                    w2_ref, b2_ref, g2_ref, be2_ref), 0.0)


def _dense_last_body(h_ref, a_ref, w1_ref, b1_ref, g1_ref, be1_ref,
                     w2_ref, b2_ref, g2_ref, be2_ref, batch_ref,
                     outh_ref, outp_ref):
    hn = _dense_core(h_ref, a_ref, w1_ref, b1_ref, g1_ref, be1_ref,
                     w2_ref, b2_ref, g2_ref, be2_ref)
    outh_ref[...] = hn
    bv = batch_ref[...]
    gi = lax.broadcasted_iota(jnp.int32, (1, G), 1)
    oh = (bv == gi).astype(jnp.float32)
    xp = lax.dot_general(oh, hn, (((0,), (0,)), ((), ())),
                         preferred_element_type=jnp.float32,
                         precision=lax.Precision.HIGHEST)
    nrm = jnp.sqrt(jnp.sum(xp * xp, axis=1, keepdims=True))
    outp_ref[...] = xp / jnp.maximum(nrm, 1e-12)


_dense_mid = pl.pallas_call(
    _dense_mid_body,
    out_shape=jax.ShapeDtypeStruct((N, D), jnp.float32),
)

_dense_last = pl.pallas_call(
    _dense_last_body,
    out_shape=(jax.ShapeDtypeStruct((N, D), jnp.float32),
               jax.ShapeDtypeStruct((G, D), jnp.float32)),
)


def kernel(params, batch, x, edge_index, edge_attr):
    atom = params['atom_table']
    bond = params['bond_table']
    layers = params['layers']
    src = edge_index[0].astype(jnp.int32)
    dst = edge_index[1].astype(jnp.int32)
    ea = edge_attr[:, 0].astype(jnp.int32)
    pad = ((0, 0), (0, EPW_P - EPW))
    idx = jnp.pad((src * BOND_VOCAB + ea).reshape(NW, EPW),
                  pad).reshape(NW, NCHUNK, C)
    dstp = jnp.pad(dst.reshape(NW, EPW), pad,
                   constant_values=N_PAD - 1).reshape(NW, NCHUNK, C)
    xi = x.astype(jnp.int32)
    batchf = batch.astype(jnp.int32).reshape(N, 1)

    h = _embed(xi, atom)
    n_layers = len(layers)
    xpool = None
    for i, p in enumerate(layers):
        msg = _msg(h, bond).reshape(N * BOND_VOCAB, D)
        aggr = _make_sc_aggr()(msg, idx, dstp)
        args = (h, aggr, p['W1'], p['b1'].reshape(1, -1),
                p['g_mlp'].reshape(1, -1), p['be_mlp'].reshape(1, -1),
                p['W2'], p['b2'].reshape(1, -1),
                p['g_bn'].reshape(1, -1), p['be_bn'].reshape(1, -1))
        if i < n_layers - 1:
            h = _dense_mid(*args)
        else:
            h, xpool = _dense_last(*args, batchf)
    return (xpool, h)
